# baseline (speedup 1.0000x reference)
import jax
import jax.numpy as jnp
from jax.experimental import pallas as pl
from jax.experimental.pallas import tpu as pltpu


def _probe_body(x_ref, o_ref):
    o_ref[0, 0] = jnp.sum(x_ref[...])


def kernel(x, y):
    out = pl.pallas_call(
        _probe_body,
        grid=(1,),
        in_specs=[pl.BlockSpec((8, 128), lambda i: (0, 0))],
        out_specs=pl.BlockSpec(memory_space=pltpu.SMEM),
        out_shape=jax.ShapeDtypeStruct((1, 1), jnp.float32),
    )(x)
    return out.reshape(())


# speedup vs baseline: 1.3625x; 1.3625x over previous
import jax
import jax.numpy as jnp
from jax.experimental import pallas as pl
from jax.experimental.pallas import tpu as pltpu

_B = 16384
_V = 1000
_K = 11468
_C = 512
_G = _B // _C


def _ohem_body(xt_ref, y_ref, o_ref, loss_sc):
    i = pl.program_id(0)
    xt = xt_ref[...]
    xm = jnp.max(xt, axis=0, keepdims=True)
    e = jnp.exp(xt - xm)
    row = jax.lax.broadcasted_iota(jnp.int32, (_V, _C), 0)
    yb = y_ref[0]
    w = jnp.where(row == yb, xt, 0.0)
    ones = jnp.ones((1, _V), jnp.float32)
    dn = (((1,), (0,)), ((), ()))
    s = jax.lax.dot_general(ones, e, dn, preferred_element_type=jnp.float32)
    xy = jax.lax.dot_general(ones, w, dn, preferred_element_type=jnp.float32)
    loss = (xm - xy) + jnp.log(s)
    loss_sc[pl.ds(i, 1), :] = loss

    @pl.when(i == _G - 1)
    def _select():
        vals = loss_sc[...]
        bits = jax.lax.bitcast_convert_type(vals, jnp.int32)

        def body(j, p):
            t = p | (jnp.int32(1) << (jnp.int32(30) - j))
            c = jnp.sum((bits >= t).astype(jnp.int32))
            return jnp.where(c >= _K, t, p)

        p = jax.lax.fori_loop(0, 31, body, jnp.int32(0))
        gt = bits > p
        c_gt = jnp.sum(gt.astype(jnp.int32))
        s_gt = jnp.sum(jnp.where(gt, vals, 0.0))
        tval = jnp.max(jnp.where(bits == p, vals, 0.0))
        total = s_gt + (jnp.int32(_K) - c_gt).astype(jnp.float32) * tval
        o_ref[0, 0] = total / jnp.float32(_K)


def kernel(x, y):
    xt = x.T
    y3 = y.astype(jnp.int32).reshape(_G, 1, _C)
    out = pl.pallas_call(
        _ohem_body,
        grid=(_G,),
        in_specs=[
            pl.BlockSpec((_V, _C), lambda i: (0, i)),
            pl.BlockSpec((1, 1, _C), lambda i: (i, 0, 0)),
        ],
        out_specs=pl.BlockSpec(memory_space=pltpu.SMEM),
        out_shape=jax.ShapeDtypeStruct((1, 1), jnp.float32),
        scratch_shapes=[pltpu.VMEM((_G, _C), jnp.float32)],
        compiler_params=pltpu.CompilerParams(dimension_semantics=("arbitrary",)),
    )(xt, y3)
    return out.reshape(())


# speedup vs baseline: 1.9064x; 1.3993x over previous
import jax
import jax.numpy as jnp
from jax.experimental import pallas as pl
from jax.experimental.pallas import tpu as pltpu

_B = 16384
_V = 1000
_K = 11468
_C = 512
_NS = 4
_G = _B // _C // _NS
_NB = _B // _C


def _ohem_body(x0, x1, x2, x3, y0, y1, y2, y3, o_ref, loss_sc):
    i = pl.program_id(0)
    ones = jnp.ones((1, _V), jnp.float32)
    dn = (((1,), (0,)), ((), ()))
    row = jax.lax.broadcasted_iota(jnp.int32, (_V, _C), 0)

    for s, (x_ref, y_ref) in enumerate(
        ((x0, y0), (x1, y1), (x2, y2), (x3, y3))
    ):
        xt = x_ref[...]
        xm = jnp.max(xt, axis=0, keepdims=True)
        e = jnp.exp(xt - xm)
        yb = y_ref[0]
        w = jnp.where(row == yb, xt, 0.0)
        s_ = jax.lax.dot_general(ones, e, dn, preferred_element_type=jnp.float32)
        xy = jax.lax.dot_general(ones, w, dn, preferred_element_type=jnp.float32)
        loss = (xm - xy) + jnp.log(s_)
        loss_sc[pl.ds(s * _G + i, 1), :] = loss

    @pl.when(i == _G - 1)
    def _select():
        vals = loss_sc[...]
        bits = jax.lax.bitcast_convert_type(vals, jnp.int32)

        def body(j, p):
            t = p | (jnp.int32(1) << (jnp.int32(30) - j))
            c = jnp.sum((bits >= t).astype(jnp.int32))
            return jnp.where(c >= _K, t, p)

        p = jax.lax.fori_loop(0, 31, body, jnp.int32(0))
        gt = bits > p
        c_gt = jnp.sum(gt.astype(jnp.int32))
        s_gt = jnp.sum(jnp.where(gt, vals, 0.0))
        tval = jnp.max(jnp.where(bits == p, vals, 0.0))
        total = s_gt + (jnp.int32(_K) - c_gt).astype(jnp.float32) * tval
        o_ref[0, 0] = total / jnp.float32(_K)


def kernel(x, y):
    xt = x.T
    yr = y.astype(jnp.int32).reshape(_NB, 1, _C)
    x_specs = [
        pl.BlockSpec((_V, _C), lambda i, s=s: (0, s * _G + i)) for s in range(_NS)
    ]
    y_specs = [
        pl.BlockSpec((1, 1, _C), lambda i, s=s: (s * _G + i, 0, 0))
        for s in range(_NS)
    ]
    out = pl.pallas_call(
        _ohem_body,
        grid=(_G,),
        in_specs=x_specs + y_specs,
        out_specs=pl.BlockSpec(memory_space=pltpu.SMEM),
        out_shape=jax.ShapeDtypeStruct((1, 1), jnp.float32),
        scratch_shapes=[pltpu.VMEM((_NB, _C), jnp.float32)],
        compiler_params=pltpu.CompilerParams(dimension_semantics=("arbitrary",)),
    )(xt, xt, xt, xt, yr, yr, yr, yr)
    return out.reshape(())
